# baseline (speedup 1.0000x reference)
import functools

import jax
import jax.numpy as jnp
from jax.experimental import pallas as pl
from jax.experimental.pallas import tpu as pltpu

B, T, DM = 1, 2048, 1024
H, HKV, DH = 16, 4, 64
TOPK = 64
NIH, IHD = 4, 32
EPS = 1.1920929e-07
SCALE = DH ** -0.5
G = H // HKV

BLK = 1024
NBLK = T // BLK
NEG = -1e30


def _rope_rms_head(xh, c, s):
    d = DH // 2
    rot = jnp.concatenate([-xh[:, d:], xh[:, :d]], axis=1)
    r = xh * c + rot * s
    return r * jax.lax.rsqrt(jnp.mean(r * r, axis=-1, keepdims=True) + EPS)


def _mono_key(x):
    b = jax.lax.bitcast_convert_type(x, jnp.int32)
    return jnp.where(b >= 0, b, b ^ jnp.int32(0x7FFFFFFF))


def _kv_kernel(x_ref, cos_ref, sin_ref, w_ref, wik_ref, k_ref, v_ref, ki_ref):
    x = x_ref[...]
    y = jnp.dot(x.astype(jnp.bfloat16), w_ref[...].astype(jnp.bfloat16),
                preferred_element_type=jnp.float32)
    c = cos_ref[...]
    s = sin_ref[...]
    for h in range(HKV):
        k_ref[:, h * DH:(h + 1) * DH] = _rope_rms_head(
            y[:, h * DH:(h + 1) * DH], c, s)
    v_ref[...] = y[:, HKV * DH:2 * HKV * DH]
    ki_ref[...] = jnp.dot(x, wik_ref[...], preferred_element_type=jnp.float32)


def _make_attn_kernel(width, row0):
    def body(x_ref, cos_ref, sin_ref, wq_ref, wiq_ref, wiw_ref, wo_ref,
             k_ref, v_ref, ki_ref, tri_ref, out_ref):
        blk = pl.program_id(0)
        x = x_ref[...]
        c = cos_ref[...]
        s = sin_ref[...]

        yq = jnp.dot(x.astype(jnp.bfloat16), wq_ref[...].astype(jnp.bfloat16),
                     preferred_element_type=jnp.float32)
        qh = [_rope_rms_head(yq[:, h * DH:(h + 1) * DH], c, s)
              for h in range(H)]

        qi = jnp.dot(x, wiq_ref[...], preferred_element_type=jnp.float32)
        wi = jnp.dot(x, wiw_ref[...], preferred_element_type=jnp.float32)
        ki = ki_ref[...]
        acc = jnp.zeros((BLK, width), jnp.float32)
        for h in range(NIH):
            raw = jax.lax.dot_general(
                qi[:, h * IHD:(h + 1) * IHD], ki,
                (((1,), (1,)), ((), ())), preferred_element_type=jnp.float32)
            acc = acc + jnp.maximum(raw, 0.0) * wi[:, h:h + 1]

        col = jax.lax.broadcasted_iota(jnp.int32, (BLK, width), 1)
        row = jax.lax.broadcasted_iota(jnp.int32, (BLK, width), 0) \
            + (row0 + blk * BLK)
        valid = col <= row
        masked = jnp.where(valid, acc, -jnp.inf)
        key = _mono_key(masked)

        ones_b = jnp.ones((width, 1), jnp.bfloat16)
        NCH = 4
        RC = BLK // NCH
        hi16 = jax.lax.shift_right_arithmetic(
            key, jnp.int32(16)).astype(jnp.int16)
        lo16 = (key & jnp.int32(0xFFFF)).astype(jnp.int16) ^ jnp.int16(-32768)

        def cnt_ge16(kc, v):
            m = (kc >= v).astype(jnp.bfloat16)
            return jnp.dot(m, ones_b, preferred_element_type=jnp.float32)

        def ok16(cnt, tc):
            return (cnt >= tc).astype(jnp.int16)

        def search16(vals_c, target_c):
            out = []
            for i in range(NCH):
                o = ok16(cnt_ge16(vals_c[i], jnp.zeros((RC, 1), jnp.int16)),
                         target_c[i])
                out.append(jnp.int16(-32768) * (jnp.int16(1) - o))
            for bit in range(14, -1, -1):
                for i in range(NCH):
                    cand = out[i] | jnp.int16(1 << bit)
                    o = ok16(cnt_ge16(vals_c[i], cand), target_c[i])
                    out[i] = out[i] | (cand & (-o))
            return out

        hi_c = [hi16[i * RC:(i + 1) * RC] for i in range(NCH)]
        topk_c = [jnp.full((RC, 1), TOPK, jnp.float32) for _ in range(NCH)]
        vhi_c = search16(hi_c, topk_c)

        lo_c, need2_c = [], []
        for i in range(NCH):
            cnt_gt_hi = jnp.dot((hi_c[i] > vhi_c[i]).astype(jnp.bfloat16),
                                ones_b, preferred_element_type=jnp.float32)
            need2_c.append(TOPK - cnt_gt_hi)
            band = hi_c[i] == vhi_c[i]
            lo_c.append(jnp.where(band, lo16[i * RC:(i + 1) * RC],
                                  jnp.int16(-32768)))
        vlo_c = search16(lo_c, need2_c)

        v64 = jnp.concatenate(
            [(vhi_c[i].astype(jnp.int32) << 16)
             | ((vlo_c[i] ^ jnp.int16(-32768)).astype(jnp.int32)
                & jnp.int32(0xFFFF))
             for i in range(NCH)], axis=0)

        gt = key > v64
        eq = key == v64
        need = TOPK - jnp.dot(gt.astype(jnp.bfloat16), ones_b,
                              preferred_element_type=jnp.float32)

        eq_b = jnp.where(eq, 1.0, 0.0).astype(jnp.bfloat16)
        pc = jnp.dot(eq_b, tri_ref[...], preferred_element_type=jnp.float32)
        sel = (gt | (eq & (pc < need))) & valid

        oh = [None] * H
        for h in range(H):
            n = h // G
            kn = k_ref[:, n * DH:(n + 1) * DH]
            vn = v_ref[:, n * DH:(n + 1) * DH]
            ve = jnp.concatenate(
                [vn, jnp.ones((width, 1), jnp.float32)], axis=1)
            sc = jax.lax.dot_general(
                (qh[h] * SCALE).astype(jnp.bfloat16), kn.astype(jnp.bfloat16),
                (((1,), (1,)), ((), ())),
                preferred_element_type=jnp.float32)
            p = jnp.where(sel, jnp.exp(sc), 0.0)
            o = jnp.dot(p.astype(jnp.bfloat16), ve.astype(jnp.bfloat16),
                        preferred_element_type=jnp.float32)
            oh[h] = o[:, :DH] / o[:, DH:DH + 1]

        out_heads = jnp.concatenate(oh, axis=1)
        out_ref[...] = jnp.dot(out_heads.astype(jnp.bfloat16),
                               wo_ref[...].astype(jnp.bfloat16),
                               preferred_element_type=jnp.float32)

    return body


@jax.jit
def kernel(x, cos, sin, Wq, Wk, Wv, Wo, Wiq, Wiw, Wik):
    x2 = x.reshape(T, DM)
    cos2 = cos.reshape(T, DH)
    sin2 = sin.reshape(T, DH)
    wkv = jnp.concatenate([Wk, Wv], axis=1)

    full = lambda shape: pl.BlockSpec(shape, lambda i: (0, 0))
    rows = lambda w: pl.BlockSpec((BLK, w), lambda i: (i, 0))

    k, v, ki = pl.pallas_call(
        _kv_kernel,
        grid=(NBLK,),
        in_specs=[rows(DM), rows(DH), rows(DH), full((DM, 2 * HKV * DH)),
                  full((DM, IHD))],
        out_specs=[rows(HKV * DH), rows(HKV * DH), rows(IHD)],
        out_shape=[
            jax.ShapeDtypeStruct((T, HKV * DH), jnp.float32),
            jax.ShapeDtypeStruct((T, HKV * DH), jnp.float32),
            jax.ShapeDtypeStruct((T, IHD), jnp.float32),
        ],
        compiler_params=pltpu.CompilerParams(
            dimension_semantics=("arbitrary",)),
    )(x2, cos2, sin2, wkv, Wik)

    tri = (jnp.arange(T)[:, None] < jnp.arange(T)[None, :]).astype(jnp.bfloat16)

    outs = []
    for p in range(NBLK):
        row0 = p * BLK
        width = (p + 1) * BLK
        o = pl.pallas_call(
            _make_attn_kernel(width, row0),
            grid=(1,),
            in_specs=[
                rows(DM), rows(DH), rows(DH),
                full((DM, H * DH)), full((DM, NIH * IHD)), full((DM, NIH)),
                full((H * DH, DM)),
                full((width, HKV * DH)), full((width, HKV * DH)),
                full((width, IHD)), full((width, width)),
            ],
            out_specs=rows(DM),
            out_shape=jax.ShapeDtypeStruct((BLK, DM), jnp.float32),
            compiler_params=pltpu.CompilerParams(
                dimension_semantics=("arbitrary",)),
        )(x2[row0:row0 + BLK], cos2[row0:row0 + BLK],
          sin2[row0:row0 + BLK], Wq, Wiq, Wiw, Wo,
          k[:width], v[:width], ki[:width], tri[:width, :width])
        outs.append(o)

    return jnp.concatenate(outs, axis=0).reshape(B, T, DM)


# speedup vs baseline: 1.1076x; 1.1076x over previous
import functools

import jax
import jax.numpy as jnp
from jax.experimental import pallas as pl
from jax.experimental.pallas import tpu as pltpu

B, T, DM = 1, 2048, 1024
H, HKV, DH = 16, 4, 64
TOPK = 64
NIH, IHD = 4, 32
EPS = 1.1920929e-07
SCALE = DH ** -0.5
G = H // HKV

BLK = 1024
NBLK = T // BLK
NEG = -1e30


def _rope_rms_head(xh, c, s):
    d = DH // 2
    rot = jnp.concatenate([-xh[:, d:], xh[:, :d]], axis=1)
    r = xh * c + rot * s
    return r * jax.lax.rsqrt(jnp.mean(r * r, axis=-1, keepdims=True) + EPS)


def _mono_key(x):
    b = jax.lax.bitcast_convert_type(x, jnp.int32)
    return jnp.where(b >= 0, b, b ^ jnp.int32(0x7FFFFFFF))


def _kv_kernel(x_ref, cos_ref, sin_ref, w_ref, wik_ref, k_ref, v_ref, ki_ref):
    x = x_ref[...]
    y = jnp.dot(x.astype(jnp.bfloat16), w_ref[...].astype(jnp.bfloat16),
                preferred_element_type=jnp.float32)
    c = cos_ref[...]
    s = sin_ref[...]
    for h in range(HKV):
        k_ref[:, h * DH:(h + 1) * DH] = _rope_rms_head(
            y[:, h * DH:(h + 1) * DH], c, s)
    v_ref[...] = y[:, HKV * DH:2 * HKV * DH]
    ki_ref[...] = jnp.dot(x, wik_ref[...], preferred_element_type=jnp.float32)


def _make_attn_kernel(width, row0):
    def body(x_ref, cos_ref, sin_ref, wq_ref, wiq_ref, wiw_ref, wo_ref,
             k_ref, v_ref, ki_ref, tri_ref, out_ref):
        blk = pl.program_id(0)
        x = x_ref[...]
        c = cos_ref[...]
        s = sin_ref[...]

        yq = jnp.dot(x.astype(jnp.bfloat16), wq_ref[...].astype(jnp.bfloat16),
                     preferred_element_type=jnp.float32)
        qh = [_rope_rms_head(yq[:, h * DH:(h + 1) * DH], c, s)
              for h in range(H)]

        qi = jnp.dot(x, wiq_ref[...], preferred_element_type=jnp.float32)
        wi = jnp.dot(x, wiw_ref[...], preferred_element_type=jnp.float32)
        ki = ki_ref[...]
        acc = jnp.zeros((BLK, width), jnp.float32)
        for h in range(NIH):
            raw = jax.lax.dot_general(
                qi[:, h * IHD:(h + 1) * IHD], ki,
                (((1,), (1,)), ((), ())), preferred_element_type=jnp.float32)
            acc = acc + jnp.maximum(raw, 0.0) * wi[:, h:h + 1]

        col = jax.lax.broadcasted_iota(jnp.int32, (BLK, width), 1)
        row = jax.lax.broadcasted_iota(jnp.int32, (BLK, width), 0) \
            + (row0 + blk * BLK)
        valid = col <= row
        masked = jnp.where(valid, acc, -jnp.inf)
        key = _mono_key(masked)

        ones_b = jnp.ones((width, 1), jnp.bfloat16)
        NCH = 4
        RC = BLK // NCH
        keys_c = [key[i * RC:(i + 1) * RC] for i in range(NCH)]

        def cnt_ge(kc, v):
            m = (kc >= v).astype(jnp.bfloat16)
            return jnp.dot(m, ones_b, preferred_element_type=jnp.float32)

        v64_c = [
            jnp.broadcast_to(
                jnp.where(cnt_ge(keys_c[i],
                                 jnp.zeros((RC, 1), jnp.int32)) >= TOPK,
                          jnp.int32(0), jnp.int32(-2147483648)),
                (RC, 1))
            for i in range(NCH)
        ]
        for bit in range(30, -1, -1):
            for i in range(NCH):
                cand = v64_c[i] | jnp.int32(1 << bit)
                v64_c[i] = jnp.where(cnt_ge(keys_c[i], cand) >= TOPK,
                                     cand, v64_c[i])
        v64 = jnp.concatenate(v64_c, axis=0)

        gt = key > v64
        eq = key == v64
        need = TOPK - jnp.dot(gt.astype(jnp.bfloat16), ones_b,
                              preferred_element_type=jnp.float32)

        eq_b = jnp.where(eq, 1.0, 0.0).astype(jnp.bfloat16)
        pc = jnp.dot(eq_b, tri_ref[...], preferred_element_type=jnp.float32)
        sel = (gt | (eq & (pc < need))) & valid

        oh = [None] * H
        for h in range(H):
            n = h // G
            kn = k_ref[:, n * DH:(n + 1) * DH]
            vn = v_ref[:, n * DH:(n + 1) * DH]
            ve = jnp.concatenate(
                [vn, jnp.ones((width, 1), jnp.float32)], axis=1)
            sc = jax.lax.dot_general(
                (qh[h] * SCALE).astype(jnp.bfloat16), kn.astype(jnp.bfloat16),
                (((1,), (1,)), ((), ())),
                preferred_element_type=jnp.float32)
            p = jnp.where(sel, jnp.exp(sc), 0.0)
            o = jnp.dot(p.astype(jnp.bfloat16), ve.astype(jnp.bfloat16),
                        preferred_element_type=jnp.float32)
            oh[h] = o[:, :DH] / o[:, DH:DH + 1]

        out_heads = jnp.concatenate(oh, axis=1)
        out_ref[...] = jnp.dot(out_heads.astype(jnp.bfloat16),
                               wo_ref[...].astype(jnp.bfloat16),
                               preferred_element_type=jnp.float32)

    return body


@jax.jit
def kernel(x, cos, sin, Wq, Wk, Wv, Wo, Wiq, Wiw, Wik):
    x2 = x.reshape(T, DM)
    cos2 = cos.reshape(T, DH)
    sin2 = sin.reshape(T, DH)
    wkv = jnp.concatenate([Wk, Wv], axis=1)

    full = lambda shape: pl.BlockSpec(shape, lambda i: (0, 0))
    rows = lambda w: pl.BlockSpec((BLK, w), lambda i: (i, 0))

    k, v, ki = pl.pallas_call(
        _kv_kernel,
        grid=(NBLK,),
        in_specs=[rows(DM), rows(DH), rows(DH), full((DM, 2 * HKV * DH)),
                  full((DM, IHD))],
        out_specs=[rows(HKV * DH), rows(HKV * DH), rows(IHD)],
        out_shape=[
            jax.ShapeDtypeStruct((T, HKV * DH), jnp.float32),
            jax.ShapeDtypeStruct((T, HKV * DH), jnp.float32),
            jax.ShapeDtypeStruct((T, IHD), jnp.float32),
        ],
        compiler_params=pltpu.CompilerParams(
            dimension_semantics=("arbitrary",)),
    )(x2, cos2, sin2, wkv, Wik)

    tri = (jnp.arange(T)[:, None] < jnp.arange(T)[None, :]).astype(jnp.bfloat16)

    outs = []
    for p in range(NBLK):
        row0 = p * BLK
        width = (p + 1) * BLK
        o = pl.pallas_call(
            _make_attn_kernel(width, row0),
            grid=(1,),
            in_specs=[
                rows(DM), rows(DH), rows(DH),
                full((DM, H * DH)), full((DM, NIH * IHD)), full((DM, NIH)),
                full((H * DH, DM)),
                full((width, HKV * DH)), full((width, HKV * DH)),
                full((width, IHD)), full((width, width)),
            ],
            out_specs=rows(DM),
            out_shape=jax.ShapeDtypeStruct((BLK, DM), jnp.float32),
            compiler_params=pltpu.CompilerParams(
                dimension_semantics=("arbitrary",)),
        )(x2[row0:row0 + BLK], cos2[row0:row0 + BLK],
          sin2[row0:row0 + BLK], Wq, Wiq, Wiw, Wo,
          k[:width], v[:width], ki[:width], tri[:width, :width])
        outs.append(o)

    return jnp.concatenate(outs, axis=0).reshape(B, T, DM)
